# baseline (speedup 1.0000x reference)
import jax
import jax.numpy as jnp
from jax import lax
from jax.experimental import pallas as pl
from jax.experimental.pallas import tpu as pltpu

K = 16
C = 2048
P = 64
B = 8
N = 64
O1 = 256
O2 = 512


def _gcn_kernel(snd_ref, w1_ref, b1_ref, L_ref, fc1_ref, bfc_ref, out_ref,
                nodes_ref, w1b_ref, fc1b_ref):
    @pl.when(pl.program_id(0) == 0)
    def _():
        w1b_ref[...] = w1_ref[...].astype(jnp.bfloat16)
        fc1b_ref[...] = fc1_ref[...].astype(jnp.bfloat16)

    nodes = nodes_ref[...]
    x = lax.dot_general(nodes, w1b_ref[...], (((1,), (1,)), ((), ())),
                        preferred_element_type=jnp.float32)
    x = jnp.maximum(x + b1_ref[...], 0.0)

    Lm = L_ref[...]
    y = jnp.concatenate(
        [jnp.dot(Lm, x[b * K:(b + 1) * K, :],
                 preferred_element_type=jnp.float32) for b in range(B)],
        axis=0)

    yr = y.reshape(B, K * O1).astype(jnp.bfloat16)
    out_ref[...] = lax.dot_general(yr, fc1b_ref[...], (((1,), (1,)), ((), ())),
                                   preferred_element_type=jnp.float32) \
        + bfc_ref[...]

    blk3 = snd_ref[...]
    blk = blk3.reshape(B * P, C)
    feat = jnp.sum(blk3, axis=2)
    ft = jnp.transpose(feat, (1, 0))

    iota = lax.broadcasted_iota(jnp.int32, (P, B), 0)
    pos_list = []
    for _ in range(K):
        m = jnp.max(ft, axis=0, keepdims=True)
        cand = jnp.where(ft == m, iota, P)
        idx = jnp.min(cand, axis=0, keepdims=True)
        ft = jnp.where(iota == idx, -jnp.inf, ft)
        r = jnp.where(idx < 8, idx >> 3, (idx >> 3) - 1)
        r = jnp.clip(r, 0, 7)
        cm = idx & 7
        c = jnp.where(cm == 0, 7, cm - 1)
        pos_list.append(r * 8 + c)
    pos = jnp.concatenate(pos_list, axis=0)

    boff = lax.broadcasted_iota(jnp.int32, (B, K), 0) * P
    tgt = (jnp.transpose(pos, (1, 0)) + boff)[:, :, None]
    colio = lax.broadcasted_iota(jnp.int32, (B, K, B * P), 2)
    ohb = (colio == tgt).astype(jnp.bfloat16).reshape(B * K, B * P)

    nodes_ref[...] = jnp.dot(ohb, blk.astype(jnp.bfloat16),
                             preferred_element_type=jnp.float32
                             ).astype(jnp.bfloat16)


def kernel(sounds, conv1_w, conv1_b, fc1_w, fc1_b, Lnorm, interpret=False):
    snd = jnp.transpose(sounds.reshape(N, C, P), (0, 2, 1))
    b1 = conv1_b.reshape(1, O1)
    bfc = fc1_b.reshape(1, O2)
    nb = N // B

    return pl.pallas_call(
        _gcn_kernel,
        grid=(nb + 1,),
        in_specs=[
            pl.BlockSpec((B, P, C), lambda i: (jnp.minimum(i, nb - 1), 0, 0)),
            pl.BlockSpec((O1, C), lambda i: (0, 0)),
            pl.BlockSpec((1, O1), lambda i: (0, 0)),
            pl.BlockSpec((K, K), lambda i: (0, 0)),
            pl.BlockSpec((O2, K * O1), lambda i: (0, 0)),
            pl.BlockSpec((1, O2), lambda i: (0, 0)),
        ],
        out_specs=pl.BlockSpec((B, O2), lambda i: (jnp.maximum(i - 1, 0), 0)),
        out_shape=jax.ShapeDtypeStruct((N, O2), jnp.float32),
        scratch_shapes=[pltpu.VMEM((B * K, C), jnp.bfloat16),
                        pltpu.VMEM((O1, C), jnp.bfloat16),
                        pltpu.VMEM((O2, K * O1), jnp.bfloat16)],
        compiler_params=pltpu.CompilerParams(
            dimension_semantics=("arbitrary",)),
        interpret=interpret,
    )(snd, conv1_w, b1, Lnorm, fc1_w, bfc)


# speedup vs baseline: 1.1712x; 1.1712x over previous
import jax
import jax.numpy as jnp
from jax import lax
from jax.experimental import pallas as pl
from jax.experimental.pallas import tpu as pltpu

K = 16
C = 2048
P = 64
B = 8
N = 64
O1 = 256
O2 = 512


def _gcn_kernel(snd_ref, w1_ref, b1_ref, L_ref, fc1_ref, bfc_ref, out_ref,
                nodes_ref, w1b_ref, y_ref):
    @pl.when(pl.program_id(0) == 0)
    def _():
        w1b_ref[...] = w1_ref[...].astype(jnp.bfloat16)

    nodes = nodes_ref[...]
    x = lax.dot_general(nodes, w1b_ref[...], (((1,), (1,)), ((), ())),
                        preferred_element_type=jnp.float32)
    x = jnp.maximum(x + b1_ref[...], 0.0)

    Lm = L_ref[...]
    y = jnp.concatenate(
        [jnp.dot(Lm, x[b * K:(b + 1) * K, :],
                 preferred_element_type=jnp.float32) for b in range(B)],
        axis=0)

    i = pl.program_id(0)
    yoff = jnp.maximum(i - 1, 0) * (B * K)
    y_ref[pl.ds(yoff, B * K), :] = y

    @pl.when(i == N // B)
    def _():
        yr = y_ref[...].reshape(N, K * O1)
        out_ref[...] = lax.dot_general(
            yr, fc1_ref[...], (((1,), (1,)), ((), ())),
            preferred_element_type=jnp.float32) + bfc_ref[...]

    blk3 = snd_ref[...]
    blk = blk3.reshape(B * P, C)
    feat = jnp.sum(blk3, axis=2)
    ft = jnp.transpose(feat, (1, 0))

    iota = lax.broadcasted_iota(jnp.int32, (P, B), 0)
    pos_list = []
    for _ in range(K):
        m = jnp.max(ft, axis=0, keepdims=True)
        cand = jnp.where(ft == m, iota, P)
        idx = jnp.min(cand, axis=0, keepdims=True)
        ft = jnp.where(iota == idx, -jnp.inf, ft)
        r = jnp.where(idx < 8, idx >> 3, (idx >> 3) - 1)
        r = jnp.clip(r, 0, 7)
        cm = idx & 7
        c = jnp.where(cm == 0, 7, cm - 1)
        pos_list.append(r * 8 + c)
    pos = jnp.concatenate(pos_list, axis=0)

    boff = lax.broadcasted_iota(jnp.int32, (B, K), 0) * P
    tgt = (jnp.transpose(pos, (1, 0)) + boff)[:, :, None]
    colio = lax.broadcasted_iota(jnp.int32, (B, K, B * P), 2)
    ohb = (colio == tgt).astype(jnp.bfloat16).reshape(B * K, B * P)

    nodes_ref[...] = jnp.dot(ohb, blk.astype(jnp.bfloat16),
                             preferred_element_type=jnp.float32
                             ).astype(jnp.bfloat16)


def kernel(sounds, conv1_w, conv1_b, fc1_w, fc1_b, Lnorm, interpret=False):
    snd = jnp.transpose(sounds.reshape(N, C, P), (0, 2, 1))
    b1 = conv1_b.reshape(1, O1)
    bfc = fc1_b.reshape(1, O2)
    nb = N // B

    return pl.pallas_call(
        _gcn_kernel,
        grid=(nb + 1,),
        in_specs=[
            pl.BlockSpec((B, P, C), lambda i: (jnp.minimum(i, nb - 1), 0, 0)),
            pl.BlockSpec((O1, C), lambda i: (0, 0)),
            pl.BlockSpec((1, O1), lambda i: (0, 0)),
            pl.BlockSpec((K, K), lambda i: (0, 0)),
            pl.BlockSpec((O2, K * O1), lambda i: (0, 0)),
            pl.BlockSpec((1, O2), lambda i: (0, 0)),
        ],
        out_specs=pl.BlockSpec((N, O2), lambda i: (0, 0)),
        out_shape=jax.ShapeDtypeStruct((N, O2), jnp.float32),
        scratch_shapes=[pltpu.VMEM((B * K, C), jnp.bfloat16),
                        pltpu.VMEM((O1, C), jnp.bfloat16),
                        pltpu.VMEM((N * K, O1), jnp.float32)],
        compiler_params=pltpu.CompilerParams(
            dimension_semantics=("arbitrary",)),
        interpret=interpret,
    )(snd, conv1_w, b1, Lnorm, fc1_w, bfc)
